# Initial kernel scaffold; baseline (speedup 1.0000x reference)
import jax
import jax.numpy as jnp
from jax.experimental import pallas as pl


def kernel(x, w1, b1, w2, b2, w3, b3):
    raise NotImplementedError("write your pallas kernel here")


# speedup vs baseline: 1.5398x; 1.5398x over previous
import jax
import jax.numpy as jnp
from jax.experimental import pallas as pl
from jax.experimental.pallas import tpu as pltpu

_IN_F, _H1_F, _H2_F = 245, 120, 84
_H1_P, _H2_P = 128, 128


def _mlp_fused_body(x_ref, w1_ref, b1_ref, w2_ref, b2_ref, w3_ref, b3_ref,
                    o_ref):
    h1 = jnp.tanh(
        jnp.dot(x_ref[...], w1_ref[...], preferred_element_type=jnp.float32)
        + b1_ref[...]
    )
    h2 = jnp.tanh(
        jnp.dot(h1, w2_ref[...], preferred_element_type=jnp.float32)
        + b2_ref[...]
    )
    h3 = jnp.sum(h2 * w3_ref[...], axis=1, keepdims=True) + b3_ref[...]
    o_ref[...] = jnp.maximum(h3, 0.0).astype(o_ref.dtype)


def _round_up(n, m):
    return ((n + m - 1) // m) * m


def kernel(x, w1, b1, w2, b2, w3, b3, *, tb=1024):
    B = x.shape[0]

    w1p = jnp.pad(w1, ((0, 0), (0, _H1_P - _H1_F)))
    b1p = jnp.pad(b1, ((0, 0), (0, _H1_P - _H1_F)))
    w2p = jnp.pad(w2, ((0, _H1_P - _H1_F), (0, _H2_P - _H2_F)))
    b2p = jnp.pad(b2, ((0, 0), (0, _H2_P - _H2_F)))
    w3t = jnp.pad(w3.T, ((0, 0), (0, _H2_P - _H2_F)))

    TB = min(tb, _round_up(B, 8))
    B_pad = _round_up(B, TB)
    xp = x if B_pad == B else jnp.pad(x, ((0, B_pad - B), (0, 0)))

    out = pl.pallas_call(
        _mlp_fused_body,
        out_shape=jax.ShapeDtypeStruct((B_pad, 1), jnp.float32),
        grid=(B_pad // TB,),
        in_specs=[
            pl.BlockSpec((TB, _IN_F), lambda i: (i, 0)),
            pl.BlockSpec((_IN_F, _H1_P), lambda i: (0, 0)),
            pl.BlockSpec((1, _H1_P), lambda i: (0, 0)),
            pl.BlockSpec((_H1_P, _H2_P), lambda i: (0, 0)),
            pl.BlockSpec((1, _H2_P), lambda i: (0, 0)),
            pl.BlockSpec((1, _H2_P), lambda i: (0, 0)),
            pl.BlockSpec((1, 1), lambda i: (0, 0)),
        ],
        out_specs=pl.BlockSpec((TB, 1), lambda i: (i, 0)),
        compiler_params=pltpu.CompilerParams(
            dimension_semantics=("parallel",)
        ),
    )(xp, w1p, b1p, w2p, b2p, w3t, b3)

    return out[:B]


# speedup vs baseline: 1.8583x; 1.2069x over previous
import jax
import jax.numpy as jnp
from jax.experimental import pallas as pl
from jax.experimental.pallas import tpu as pltpu

_IN_F, _H1_F, _H2_F = 245, 120, 84
_H1_P, _H2_P = 128, 128


def _mlp_fused_body(x_ref, w1_ref, b1_ref, w2_ref, b2_ref, w3_ref, b3_ref,
                    o_ref):
    h1 = jnp.tanh(
        jnp.dot(x_ref[...], w1_ref[...], preferred_element_type=jnp.float32)
        + b1_ref[...]
    )
    h2 = jnp.tanh(
        jnp.dot(h1, w2_ref[...], preferred_element_type=jnp.float32)
        + b2_ref[...]
    )
    z = jax.lax.dot_general(
        w3_ref[...], h2, (((1,), (1,)), ((), ())),
        preferred_element_type=jnp.float32,
    )
    y = jnp.maximum(z + b3_ref[...], 0.0)
    o_ref[...] = y.reshape(o_ref.shape).astype(o_ref.dtype)


def _round_up(n, m):
    return ((n + m - 1) // m) * m


def kernel(x, w1, b1, w2, b2, w3, b3, *, tb=1024):
    B = x.shape[0]

    w1p = jnp.pad(w1, ((0, 0), (0, _H1_P - _H1_F)))
    b1p = jnp.pad(b1, ((0, 0), (0, _H1_P - _H1_F)))
    w2p = jnp.pad(w2, ((0, _H1_P - _H1_F), (0, _H2_P - _H2_F)))
    b2p = jnp.pad(b2, ((0, 0), (0, _H2_P - _H2_F)))
    w3t = jnp.pad(w3.T, ((0, 0), (0, _H2_P - _H2_F)))

    TB = min(tb, _round_up(B, 128))
    B_pad = _round_up(B, TB)
    xp = x if B_pad == B else jnp.pad(x, ((0, B_pad - B), (0, 0)))

    out = pl.pallas_call(
        _mlp_fused_body,
        out_shape=jax.ShapeDtypeStruct((B_pad // 128, 128), jnp.float32),
        grid=(B_pad // TB,),
        in_specs=[
            pl.BlockSpec((TB, _IN_F), lambda i: (i, 0)),
            pl.BlockSpec((_IN_F, _H1_P), lambda i: (0, 0)),
            pl.BlockSpec((1, _H1_P), lambda i: (0, 0)),
            pl.BlockSpec((_H1_P, _H2_P), lambda i: (0, 0)),
            pl.BlockSpec((1, _H2_P), lambda i: (0, 0)),
            pl.BlockSpec((1, _H2_P), lambda i: (0, 0)),
            pl.BlockSpec((1, 1), lambda i: (0, 0)),
        ],
        out_specs=pl.BlockSpec((TB // 128, 128), lambda i: (i, 0)),
        compiler_params=pltpu.CompilerParams(
            dimension_semantics=("parallel",)
        ),
    )(xp, w1p, b1p, w2p, b2p, w3t, b3)

    return out.reshape(B_pad, 1)[:B]
